# baseline (speedup 1.0000x reference)
import functools

import jax
import jax.numpy as jnp
from jax import lax
from jax.experimental import pallas as pl
from jax.experimental.pallas import tpu as pltpu
from jax.experimental.pallas import tpu_sc as plsc

_NC, _NS = 2, 16
_NW = _NC * _NS


def kernel(x, embedding):
    B, S = x.shape
    D = embedding.shape[1]
    rows_w = S // _NW
    CH = 32
    n_ch = rows_w // CH

    mesh = plsc.VectorSubcoreMesh(core_axis_name="c", subcore_axis_name="s")

    @functools.partial(
        pl.kernel,
        out_type=jax.ShapeDtypeStruct((B, S, D), jnp.float32),
        mesh=mesh,
        scratch_types=[
            pltpu.VMEM((CH, D), jnp.float32),
        ],
    )
    def sc_copy(emb_hbm, out_hbm, buf):
        wid = lax.axis_index("s") * _NC + lax.axis_index("c")
        base = wid * rows_w

        def body(i, carry):
            r = base + i * CH
            pltpu.sync_copy(emb_hbm.at[pl.ds(r, CH)], buf)
            for b in range(B):
                pltpu.sync_copy(buf, out_hbm.at[b, pl.ds(r, CH)])
            return carry

        lax.fori_loop(0, n_ch, body, 0)

    return sc_copy(embedding[:S])


# speedup vs baseline: 1.0249x; 1.0249x over previous
import functools

import jax
import jax.numpy as jnp
from jax import lax
from jax.experimental import pallas as pl
from jax.experimental.pallas import tpu as pltpu
from jax.experimental.pallas import tpu_sc as plsc

_NC, _NS = 2, 16
_NW = _NC * _NS


def kernel(x, embedding):
    B, S = x.shape
    D = embedding.shape[1]
    rows_w = S // _NW
    CH = 32
    n_ch = rows_w // CH

    mesh = plsc.VectorSubcoreMesh(core_axis_name="c", subcore_axis_name="s")

    @functools.partial(
        pl.kernel,
        out_type=jax.ShapeDtypeStruct((B, S, D), jnp.float32),
        mesh=mesh,
        scratch_types=[
            pltpu.VMEM((CH, D), jnp.float32),
            pltpu.VMEM((CH, D), jnp.float32),
            pltpu.SemaphoreType.DMA,
            pltpu.SemaphoreType.DMA,
            pltpu.SemaphoreType.DMA,
            pltpu.SemaphoreType.DMA,
        ],
    )
    def sc_copy(emb_hbm, out_hbm, buf0, buf1, rs0, rs1, ws0, ws1):
        wid = lax.axis_index("s") * _NC + lax.axis_index("c")
        base = wid * rows_w
        bufs, rsems, wsems = (buf0, buf1), (rs0, rs1), (ws0, ws1)

        def read(i):
            r = base + i * CH
            return pltpu.make_async_copy(
                emb_hbm.at[pl.ds(r, CH)], bufs[i % 2], rsems[i % 2])

        def writes(i):
            r = base + i * CH
            return [
                pltpu.make_async_copy(
                    bufs[i % 2], out_hbm.at[b, pl.ds(r, CH)], wsems[i % 2])
                for b in range(B)
            ]

        read(0).start()
        read(1).start()
        for i in range(n_ch):
            read(i).wait()
            for w in writes(i):
                w.start()
            if i + 2 < n_ch:
                for w in writes(i):
                    w.wait()
                read(i + 2).start()
        for i in (n_ch - 2, n_ch - 1):
            for w in writes(i):
                w.wait()

    return sc_copy(embedding[:S])
